# baseline (speedup 1.0000x reference)
import functools

import jax
import jax.numpy as jnp
from jax.experimental import pallas as pl
from jax.experimental.pallas import tpu as pltpu


def _interp_matrix(n_pix: int, n_cell: int):
    p = (jnp.arange(n_pix, dtype=jnp.float32) + 0.5) * (n_cell / n_pix)
    cell = jnp.arange(n_cell, dtype=jnp.float32)
    w = jnp.maximum(1.0 - jnp.abs(cell[None, :] + 0.5 - p[:, None]), 0.0)
    w = w.at[:, 0].add(jnp.maximum(0.5 - p, 0.0))
    w = w.at[:, -1].add(jnp.maximum(p - (n_cell - 0.5), 0.0))
    return w


def _bsa_kernel(grid_ref, ay_ref, axk_ref, guide_ref, img_ref, out_ref,
                *, n_in, n_out, D, Gw, TH, W):
    cs = n_in + 1
    g = grid_ref[0]
    ay = ay_ref[...]
    t1 = jnp.dot(ay, g, preferred_element_type=jnp.float32)

    gz = guide_ref[0] * float(D)
    ws = []
    for k in range(D):
        w = jnp.maximum(1.0 - jnp.abs(gz - (k + 0.5)), 0.0)
        if k == 0:
            w = w + jnp.maximum(0.5 - gz, 0.0)
        if k == D - 1:
            w = w + jnp.maximum(gz - (D - 0.5), 0.0)
        ws.append(w)

    axk = axk_ref[...]
    img = img_ref[0]
    t1_bf = t1.astype(jnp.bfloat16)
    C = n_out * cs
    t2 = [jnp.dot(t1_bf[:, c * D * Gw:(c + 1) * D * Gw], axk,
                  preferred_element_type=jnp.float32) for c in range(C)]
    coeff = [None] * C
    for k in range(D):
        wk = ws[k]
        sl = slice(k * W, (k + 1) * W)
        for c in range(C):
            term = wk * t2[c][:, sl]
            coeff[c] = term if coeff[c] is None else coeff[c] + term
    for o in range(n_out):
        acc = coeff[o * cs + n_in]
        for i in range(n_in):
            acc = acc + coeff[o * cs + i] * img[i]
        out_ref[0, o] = acc


def kernel(grid, guide, image):
    B, C, D, Gh, Gw = grid.shape
    _, n_in, H, W = image.shape
    cs = n_in + 1
    n_out = C // cs
    TH = 64

    ay = _interp_matrix(H, Gh)
    axt = _interp_matrix(W, Gw).T
    axk = jnp.kron(jnp.eye(D, dtype=jnp.float32), axt).astype(jnp.bfloat16)
    grid_r = jnp.transpose(grid, (0, 3, 1, 2, 4)).reshape(B, Gh, C * D * Gw)

    f = functools.partial(_bsa_kernel, n_in=n_in, n_out=n_out, D=D, Gw=Gw,
                          TH=TH, W=W)
    out = pl.pallas_call(
        f,
        grid=(B, H // TH),
        in_specs=[
            pl.BlockSpec((1, Gh, C * D * Gw), lambda b, h: (b, 0, 0)),
            pl.BlockSpec((TH, Gh), lambda b, h: (h, 0)),
            pl.BlockSpec((D * Gw, D * W), lambda b, h: (0, 0)),
            pl.BlockSpec((1, TH, W), lambda b, h: (b, h, 0)),
            pl.BlockSpec((1, n_in, TH, W), lambda b, h: (b, 0, h, 0)),
        ],
        out_specs=pl.BlockSpec((1, n_out, TH, W), lambda b, h: (b, 0, h, 0)),
        out_shape=jax.ShapeDtypeStruct((B, n_out, H, W), jnp.float32),
    )(grid_r, ay, axk, guide, image)
    return out


# speedup vs baseline: 1.0583x; 1.0583x over previous
import functools

import jax
import jax.numpy as jnp
from jax.experimental import pallas as pl


def _interp_matrix(n_pix: int, n_cell: int):
    p = (jnp.arange(n_pix, dtype=jnp.float32) + 0.5) * (n_cell / n_pix)
    cell = jnp.arange(n_cell, dtype=jnp.float32)
    w = jnp.maximum(1.0 - jnp.abs(cell[None, :] + 0.5 - p[:, None]), 0.0)
    w = w.at[:, 0].add(jnp.maximum(0.5 - p, 0.0))
    w = w.at[:, -1].add(jnp.maximum(p - (n_cell - 0.5), 0.0))
    return w


def _bsa_kernel(grid_ref, ay_ref, axk_ref, guide_ref, img_ref, out_ref,
                *, n_in, n_out, D, Gw, TH, W):
    cs = n_in + 1
    g = grid_ref[0]
    ay = ay_ref[...]
    t1 = jnp.dot(ay, g, preferred_element_type=jnp.float32)

    gz = guide_ref[0] * float(D)
    ws = []
    for k in range(D):
        w = jnp.maximum(1.0 - jnp.abs(gz - (k + 0.5)), 0.0)
        if k == 0:
            w = w + jnp.maximum(0.5 - gz, 0.0)
        if k == D - 1:
            w = w + jnp.maximum(gz - (D - 0.5), 0.0)
        ws.append(w)

    axk = axk_ref[...]
    img = img_ref[0]
    t1_bf = t1.astype(jnp.bfloat16)
    for o in range(n_out):
        acc = None
        for i in range(cs):
            c = o * cs + i
            t1c = t1_bf[:, c * D * Gw:(c + 1) * D * Gw]
            t2 = jnp.dot(t1c, axk,
                         preferred_element_type=jnp.float32)
            coeff = ws[0] * t2[:, 0:W]
            for k in range(1, D):
                coeff = coeff + ws[k] * t2[:, k * W:(k + 1) * W]
            term = coeff if i == n_in else coeff * img[i]
            acc = term if acc is None else acc + term
        out_ref[0, o] = acc


def kernel(grid, guide, image):
    B, C, D, Gh, Gw = grid.shape
    _, n_in, H, W = image.shape
    cs = n_in + 1
    n_out = C // cs
    TH = 128

    ay = _interp_matrix(H, Gh)
    axt = _interp_matrix(W, Gw).T
    axk = jnp.kron(jnp.eye(D, dtype=jnp.float32), axt).astype(jnp.bfloat16)
    grid_r = jnp.transpose(grid, (0, 3, 1, 2, 4)).reshape(B, Gh, C * D * Gw)

    f = functools.partial(_bsa_kernel, n_in=n_in, n_out=n_out, D=D, Gw=Gw,
                          TH=TH, W=W)
    out = pl.pallas_call(
        f,
        grid=(B, H // TH),
        in_specs=[
            pl.BlockSpec((1, Gh, C * D * Gw), lambda b, h: (b, 0, 0)),
            pl.BlockSpec((TH, Gh), lambda b, h: (h, 0)),
            pl.BlockSpec((D * Gw, D * W), lambda b, h: (0, 0)),
            pl.BlockSpec((1, TH, W), lambda b, h: (b, h, 0)),
            pl.BlockSpec((1, n_in, TH, W), lambda b, h: (b, 0, h, 0)),
        ],
        out_specs=pl.BlockSpec((1, n_out, TH, W), lambda b, h: (b, 0, h, 0)),
        out_shape=jax.ShapeDtypeStruct((B, n_out, H, W), jnp.float32),
    )(grid_r, ay, axk, guide, image)
    return out


# speedup vs baseline: 1.0814x; 1.0219x over previous
import functools

import jax
import jax.numpy as jnp
from jax.experimental import pallas as pl


def _interp_matrix(n_pix: int, n_cell: int):
    p = (jnp.arange(n_pix, dtype=jnp.float32) + 0.5) * (n_cell / n_pix)
    cell = jnp.arange(n_cell, dtype=jnp.float32)
    w = jnp.maximum(1.0 - jnp.abs(cell[None, :] + 0.5 - p[:, None]), 0.0)
    w = w.at[:, 0].add(jnp.maximum(0.5 - p, 0.0))
    w = w.at[:, -1].add(jnp.maximum(p - (n_cell - 0.5), 0.0))
    return w


def _bsa_kernel(grid_ref, axk_ref, guide_ref, img_ref, out_ref,
                *, n_in, n_out, D, Gh, Gw, TH, H, W):
    cs = n_in + 1
    g = grid_ref[0]
    h = pl.program_id(1)
    row = (jax.lax.broadcasted_iota(jnp.int32, (TH, Gh), 0)
           + h * TH).astype(jnp.float32)
    py = (row + 0.5) * (Gh / H)
    cell = jax.lax.broadcasted_iota(jnp.int32, (TH, Gh), 1).astype(jnp.float32)
    ay = jnp.maximum(1.0 - jnp.abs(cell + 0.5 - py), 0.0)
    ay = ay + jnp.where(cell == 0.0, jnp.maximum(0.5 - py, 0.0), 0.0)
    ay = ay + jnp.where(cell == Gh - 1.0,
                        jnp.maximum(py - (Gh - 0.5), 0.0), 0.0)
    t1 = jnp.dot(ay, g, preferred_element_type=jnp.float32)

    gz = guide_ref[0] * float(D)
    ws = []
    for k in range(D):
        w = jnp.maximum(1.0 - jnp.abs(gz - (k + 0.5)), 0.0)
        if k == 0:
            w = w + jnp.maximum(0.5 - gz, 0.0)
        if k == D - 1:
            w = w + jnp.maximum(gz - (D - 0.5), 0.0)
        ws.append(w)

    axk = axk_ref[...]
    img = img_ref[0]
    t1_bf = t1.astype(jnp.bfloat16)
    for o in range(n_out):
        acc = None
        for i in range(cs):
            c = o * cs + i
            t1c = t1_bf[:, c * D * Gw:(c + 1) * D * Gw]
            t2 = jnp.dot(t1c, axk,
                         preferred_element_type=jnp.float32)
            coeff = ws[0] * t2[:, 0:W]
            for k in range(1, D):
                coeff = coeff + ws[k] * t2[:, k * W:(k + 1) * W]
            term = coeff if i == n_in else coeff * img[i]
            acc = term if acc is None else acc + term
        out_ref[0, o] = acc


def kernel(grid, guide, image):
    B, C, D, Gh, Gw = grid.shape
    _, n_in, H, W = image.shape
    cs = n_in + 1
    n_out = C // cs
    TH = 128

    axt = _interp_matrix(W, Gw).T
    axk = jnp.kron(jnp.eye(D, dtype=jnp.float32), axt).astype(jnp.bfloat16)
    grid_r = jnp.transpose(grid, (0, 3, 1, 2, 4)).reshape(B, Gh, C * D * Gw)

    f = functools.partial(_bsa_kernel, n_in=n_in, n_out=n_out, D=D, Gh=Gh,
                          Gw=Gw, TH=TH, H=H, W=W)
    out = pl.pallas_call(
        f,
        grid=(B, H // TH),
        in_specs=[
            pl.BlockSpec((1, Gh, C * D * Gw), lambda b, h: (b, 0, 0)),
            pl.BlockSpec((D * Gw, D * W), lambda b, h: (0, 0)),
            pl.BlockSpec((1, TH, W), lambda b, h: (b, h, 0)),
            pl.BlockSpec((1, n_in, TH, W), lambda b, h: (b, 0, h, 0)),
        ],
        out_specs=pl.BlockSpec((1, n_out, TH, W), lambda b, h: (b, 0, h, 0)),
        out_shape=jax.ShapeDtypeStruct((B, n_out, H, W), jnp.float32),
    )(grid_r, axk, guide, image)
    return out
